# baseline (speedup 1.0000x reference)
import functools

import jax
import jax.numpy as jnp
from jax import lax
from jax.experimental import pallas as pl
from jax.experimental.pallas import tpu as pltpu
from jax.experimental.pallas import tpu_sc as plsc

_M = 8192
_N = 16384
_C = 128
_NQ = 128
_NB = 2048

_VS = (0.05, 0.05, 0.1)
_OFF = (0.05, 0.05, 0.1)

_NC = 2
_NS = 16
_B = 3 * _N
_BW = _B // (_NC * _NS)
_BCH = 512


def _nn_kernel(q_ref, xiT_ref, aux_ref):
    xiT = xiT_ref[...].astype(jnp.float32)
    kb = xiT[0:1, :]
    kx = (xiT[3:4, :] * _VS[0] + _OFF[0]) + 0.5 * _VS[0]
    ky = (xiT[2:3, :] * _VS[1] + _OFF[1]) + 0.5 * _VS[1]
    kz = (xiT[1:2, :] * _VS[2] + _OFF[2]) + 0.5 * _VS[2]
    kk = ((kb * kb + kx * kx) + ky * ky) + kz * kz
    kT = jnp.concatenate([kb, kx, ky, kz], axis=0)

    q = q_ref[...]
    q0 = q[:, 0:1]
    q1 = q[:, 1:2]
    q2 = q[:, 2:3]
    q3 = q[:, 3:4]
    qq = ((q0 * q0 + q1 * q1) + q2 * q2) + q3 * q3

    cross = jnp.dot(q, kT, preferred_element_type=jnp.float32)
    d2 = jnp.maximum((qq + kk) - 2.0 * cross, 0.0)

    big = jnp.float32(1e30)
    bigl = jnp.float32(_M)
    lane = jax.lax.broadcasted_iota(jnp.int32, (_NQ, _M), 1).astype(jnp.float32)

    m0 = jnp.min(d2, axis=1, keepdims=True)
    e0 = d2 == m0
    i0 = jnp.min(jnp.where(e0, lane, bigl), axis=1, keepdims=True)
    d2 = jnp.where(e0, big, d2)
    m1 = jnp.min(d2, axis=1, keepdims=True)
    e1 = d2 == m1
    i1 = jnp.min(jnp.where(e1, lane, bigl), axis=1, keepdims=True)
    d2 = jnp.where(e1, big, d2)
    m2 = jnp.min(d2, axis=1, keepdims=True)
    e2 = d2 == m2
    i2 = jnp.min(jnp.where(e2, lane, bigl), axis=1, keepdims=True)

    r0 = 1.0 / (m0 + 1e-8)
    r1 = 1.0 / (m1 + 1e-8)
    r2 = 1.0 / (m2 + 1e-8)
    norm = (r0 + r1) + r2
    w0 = r0 / norm
    w1 = r1 / norm
    w2 = r2 / norm

    aux_ref[...] = jnp.concatenate(
        [i0, i1, i2, w0, w1, w2, jnp.zeros((_NQ, _C - 6), jnp.float32)],
        axis=1)


def _sc_gather_kernel(table_hbm, idx_hbm, out_hbm, idx_v, rows_v, sem):
    wid = lax.axis_index("s") * _NC + lax.axis_index("c")
    base = wid * _BW
    for c in range(_BW // _BCH):
        off = base + c * _BCH
        pltpu.sync_copy(idx_hbm.at[pl.ds(off, _BCH)], idx_v)
        pltpu.async_copy(table_hbm.at[idx_v], rows_v, sem).wait()
        pltpu.sync_copy(rows_v, out_hbm.at[pl.ds(off, _BCH)])


def _combine_kernel(g0_ref, g1_ref, g2_ref, aux_ref, out_ref):
    aux = aux_ref[...]
    out_ref[...] = (g0_ref[...] * aux[:, 3:4]
                    + g1_ref[...] * aux[:, 4:5]
                    + g2_ref[...] * aux[:, 5:6])


def kernel(x_features, x_indices, points_mean):
    xiT = x_indices.astype(jnp.int32).T

    aux = pl.pallas_call(
        _nn_kernel,
        grid=(_N // _NQ,),
        in_specs=[
            pl.BlockSpec((_NQ, 4), lambda i: (i, 0)),
            pl.BlockSpec((4, _M), lambda i: (0, 0)),
        ],
        out_specs=pl.BlockSpec((_NQ, _C), lambda i: (i, 0)),
        out_shape=jax.ShapeDtypeStruct((_N, _C), jnp.float32),
    )(points_mean, xiT)

    idx_flat = aux[:, :3].astype(jnp.int32).T.reshape(_B)

    mesh = plsc.VectorSubcoreMesh(core_axis_name="c", subcore_axis_name="s")
    gathered = pl.kernel(
        _sc_gather_kernel,
        mesh=mesh,
        out_type=jax.ShapeDtypeStruct((_B, _C), jnp.float32),
        scratch_types=[
            pltpu.VMEM((_BCH,), jnp.int32),
            pltpu.VMEM((_BCH, _C), jnp.float32),
            pltpu.SemaphoreType.DMA,
        ],
    )(x_features, idx_flat)

    nblk = _N // _NB
    out = pl.pallas_call(
        _combine_kernel,
        grid=(nblk,),
        in_specs=[
            pl.BlockSpec((_NB, _C), lambda i: (i, 0)),
            pl.BlockSpec((_NB, _C), lambda i: (i + nblk, 0)),
            pl.BlockSpec((_NB, _C), lambda i: (i + 2 * nblk, 0)),
            pl.BlockSpec((_NB, _C), lambda i: (i, 0)),
        ],
        out_specs=pl.BlockSpec((_NB, _C), lambda i: (i, 0)),
        out_shape=jax.ShapeDtypeStruct((_N, _C), jnp.float32),
    )(gathered, gathered, gathered, aux)
    return out


# speedup vs baseline: 2.5864x; 2.5864x over previous
import jax
import jax.numpy as jnp
from jax.experimental import pallas as pl

_M = 8192
_N = 16384
_C = 128
_NQ = 128

_VS = (0.05, 0.05, 0.1)
_OFF = (0.05, 0.05, 0.1)


def _nn_interp_kernel(q_ref, xiT_ref, feat_ref, out_ref):
    xiT = xiT_ref[...].astype(jnp.float32)
    kb = xiT[0:1, :]
    kx = (xiT[3:4, :] * _VS[0] + _OFF[0]) + 0.5 * _VS[0]
    ky = (xiT[2:3, :] * _VS[1] + _OFF[1]) + 0.5 * _VS[1]
    kz = (xiT[1:2, :] * _VS[2] + _OFF[2]) + 0.5 * _VS[2]
    kk = ((kb * kb + kx * kx) + ky * ky) + kz * kz
    kT = jnp.concatenate([kb, kx, ky, kz], axis=0)

    q = q_ref[...]
    q0 = q[:, 0:1]
    q1 = q[:, 1:2]
    q2 = q[:, 2:3]
    q3 = q[:, 3:4]
    qq = ((q0 * q0 + q1 * q1) + q2 * q2) + q3 * q3

    cross = jnp.dot(q, kT, preferred_element_type=jnp.float32)
    d2 = jnp.maximum((qq + kk) - 2.0 * cross, 0.0)

    big = jnp.float32(1e30)

    m0 = jnp.min(d2, axis=1, keepdims=True)
    e0 = d2 == m0
    d2 = jnp.where(e0, big, d2)
    m1 = jnp.min(d2, axis=1, keepdims=True)
    e1 = d2 == m1
    d2 = jnp.where(e1, big, d2)
    m2 = jnp.min(d2, axis=1, keepdims=True)
    e2 = d2 == m2

    r0 = 1.0 / (m0 + 1e-8)
    r1 = 1.0 / (m1 + 1e-8)
    r2 = 1.0 / (m2 + 1e-8)

    norm = (r0 + r1) + r2
    w0 = r0 / norm
    w1 = r1 / norm
    w2 = r2 / norm

    zero = jnp.zeros((), jnp.float32)
    w = jnp.where(e0, w0, jnp.where(e1, w1, jnp.where(e2, w2, zero)))

    out_ref[...] = jnp.dot(w, feat_ref[...],
                           preferred_element_type=jnp.float32)


def kernel(x_features, x_indices, points_mean):
    xiT = x_indices.astype(jnp.int32).T

    grid = (_N // _NQ,)
    out = pl.pallas_call(
        _nn_interp_kernel,
        grid=grid,
        in_specs=[
            pl.BlockSpec((_NQ, 4), lambda i: (i, 0)),
            pl.BlockSpec((4, _M), lambda i: (0, 0)),
            pl.BlockSpec((_M, _C), lambda i: (0, 0)),
        ],
        out_specs=pl.BlockSpec((_NQ, _C), lambda i: (i, 0)),
        out_shape=jax.ShapeDtypeStruct((_N, _C), jnp.float32),
    )(points_mean, xiT, x_features)
    return out
